# Initial kernel scaffold; baseline (speedup 1.0000x reference)
import jax
import jax.numpy as jnp
from jax.experimental import pallas as pl


def kernel(x, edge_index, batch, W1, b1, W2, b2, Wl, bl):
    raise NotImplementedError("write your pallas kernel here")


# speedup vs baseline: 4.6985x; 4.6985x over previous
import functools

import jax
import jax.numpy as jnp
from jax import lax
from jax.experimental import pallas as pl
from jax.experimental.pallas import tpu as pltpu
from jax.experimental.pallas import tpu_sc as plsc

N = 50000
E = 800000
D = 128
H = 64
C = 4
G = 128

HH = H // 2
HQ = H // 4
R = 400
NB = N // R

NP = 50176
STRIPE0 = NP // 16
EPC = E // 2
EPT0 = EPC // 16
K0C = 40
CH0 = EPT0 // K0C

NPA = 50048
EPT = E // 16
KE = 80
CH = EPT // KE
STRIPE = NPA // 16

_sc_mesh = plsc.VectorSubcoreMesh(core_axis_name="c", subcore_axis_name="s")


@functools.partial(
    pl.kernel,
    mesh=_sc_mesh,
    out_type=[
        jax.ShapeDtypeStruct((NP,), jnp.float32),
        jax.ShapeDtypeStruct((NP,), jnp.float32),
    ],
    scratch_types=[
        pltpu.VMEM((K0C,), jnp.int32),
        pltpu.VMEM((K0C,), jnp.float32),
        pltpu.VMEM((STRIPE0,), jnp.float32),
        pltpu.VMEM_SHARED((NP,), jnp.float32),
    ],
    compiler_params=pltpu.CompilerParams(use_tc_tiling_on_sc=False),
)
def _deg_kernel(dst_hbm, ones_hbm, deg0_hbm, deg1_hbm, idx_v, ones_v, stage, acc):
    c = lax.axis_index("c")
    s = lax.axis_index("s")

    zero16 = jnp.zeros((16,), jnp.float32)

    def zbody(r, carry):
        stage[pl.ds(r * 16, 16)] = zero16
        return carry

    lax.fori_loop(0, STRIPE0 // 16, zbody, 0)
    pltpu.sync_copy(stage, acc.at[pl.ds(s * STRIPE0, STRIPE0)])
    pltpu.sync_copy(ones_hbm, ones_v)
    plsc.subcore_barrier()

    def body(i, carry):
        base = c * EPC + s * EPT0 + i * K0C
        pltpu.sync_copy(dst_hbm.at[pl.ds(base, K0C)], idx_v)
        pltpu.sync_copy(ones_v, acc.at[idx_v], add=True)
        return carry

    lax.fori_loop(0, CH0, body, 0)
    plsc.subcore_barrier()
    pltpu.sync_copy(acc.at[pl.ds(s * STRIPE0, STRIPE0)], stage)

    @pl.when(c == 0)
    def _():
        pltpu.sync_copy(stage, deg0_hbm.at[pl.ds(s * STRIPE0, STRIPE0)])

    @pl.when(c == 1)
    def _():
        pltpu.sync_copy(stage, deg1_hbm.at[pl.ds(s * STRIPE0, STRIPE0)])


@functools.partial(
    pl.kernel,
    mesh=_sc_mesh,
    out_type=[jax.ShapeDtypeStruct((NPA, HQ), jnp.float32) for _ in range(4)],
    scratch_types=[
        pltpu.VMEM((KE,), jnp.int32),
        pltpu.VMEM((KE,), jnp.int32),
        pltpu.VMEM((KE, HQ), jnp.float32),
        pltpu.VMEM((STRIPE, HQ), jnp.float32),
        pltpu.VMEM_SHARED((NPA, HQ), jnp.float32),
        pltpu.SemaphoreType.DMA,
    ],
    compiler_params=pltpu.CompilerParams(use_tc_tiling_on_sc=False),
)
def _agg_kernel(yq0_hbm, yq1_hbm, yq2_hbm, yq3_hbm, src_hbm, dst_hbm,
                out0_hbm, out1_hbm, out2_hbm, out3_hbm,
                idx_s, idx_d, rows, stage, acc, sem):
    c = lax.axis_index("c")
    s = lax.axis_index("s")
    ys = (yq0_hbm, yq1_hbm, yq2_hbm, yq3_hbm)
    outs = (out0_hbm, out1_hbm, out2_hbm, out3_hbm)

    zero16 = jnp.zeros((16,), jnp.float32)

    for h in range(2):
        def zbody(r, carry):
            stage[r, pl.ds(0, 16)] = zero16
            return carry

        lax.fori_loop(0, STRIPE, zbody, 0)
        pltpu.sync_copy(stage, acc.at[pl.ds(s * STRIPE, STRIPE), :])
        plsc.subcore_barrier()

        def body(i, carry):
            base = s * EPT + i * KE
            pltpu.sync_copy(src_hbm.at[pl.ds(base, KE)], idx_s)
            pltpu.sync_copy(dst_hbm.at[pl.ds(base, KE)], idx_d)

            @pl.when(c == 0)
            def _():
                pltpu.async_copy(ys[h].at[idx_s], rows, sem).wait()

            @pl.when(c == 1)
            def _():
                pltpu.async_copy(ys[2 + h].at[idx_s], rows, sem).wait()

            pltpu.sync_copy(rows, acc.at[idx_d], add=True)
            return carry

        lax.fori_loop(0, CH, body, 0)
        plsc.subcore_barrier()
        pltpu.sync_copy(acc.at[pl.ds(s * STRIPE, STRIPE), :], stage)

        @pl.when(c == 0)
        def _():
            pltpu.sync_copy(stage, outs[h].at[pl.ds(s * STRIPE, STRIPE), :])

        @pl.when(c == 1)
        def _():
            pltpu.sync_copy(stage, outs[2 + h].at[pl.ds(s * STRIPE, STRIPE), :])


def _k1_body(x_ref, w1_ref, p0_ref, p1_ref, dinv_ref, *yq_refs):
    xw = jnp.dot(x_ref[...], w1_ref[...], preferred_element_type=jnp.float32)
    dv = lax.rsqrt(1.0 + p0_ref[0, 0, :] + p1_ref[0, 0, :])
    dinv_ref[0, 0, :] = dv
    y = xw * dv.reshape(R, 1)
    for q in range(4):
        yq_refs[q][...] = y[:, q * HQ:(q + 1) * HQ]


_k1 = pl.pallas_call(
    _k1_body,
    grid=(NB,),
    in_specs=[
        pl.BlockSpec((R, D), lambda i: (i, 0)),
        pl.BlockSpec((D, H), lambda i: (0, 0)),
        pl.BlockSpec((1, 1, R), lambda i: (i, 0, 0)),
        pl.BlockSpec((1, 1, R), lambda i: (i, 0, 0)),
    ],
    out_specs=[pl.BlockSpec((1, 1, R), lambda i: (i, 0, 0))]
    + [pl.BlockSpec((R, HQ), lambda i: (i, 0)) for _ in range(4)],
    out_shape=[jax.ShapeDtypeStruct((NB, 1, R), jnp.float32)]
    + [jax.ShapeDtypeStruct((N, HQ), jnp.float32) for _ in range(4)],
)


def _k3_body(a0, a1, a2, a3, y0, y1, y2, y3, dinv_ref, b1_ref, w2_ref,
             *yq_refs):
    dv = dinv_ref[0, 0, :].reshape(R, 1)
    h = jnp.concatenate(
        [a0[...] + y0[...], a1[...] + y1[...],
         a2[...] + y2[...], a3[...] + y3[...]], axis=1)
    h = jnp.maximum(h * dv + b1_ref[...], 0.0)
    xw = jnp.dot(h, w2_ref[...], preferred_element_type=jnp.float32)
    yy = xw * dv
    for q in range(4):
        yq_refs[q][...] = yy[:, q * HQ:(q + 1) * HQ]


_k3 = pl.pallas_call(
    _k3_body,
    grid=(NB,),
    in_specs=[pl.BlockSpec((R, HQ), lambda i: (i, 0)) for _ in range(8)]
    + [
        pl.BlockSpec((1, 1, R), lambda i: (i, 0, 0)),
        pl.BlockSpec((1, H), lambda i: (0, 0)),
        pl.BlockSpec((H, H), lambda i: (0, 0)),
    ],
    out_specs=[pl.BlockSpec((R, HQ), lambda i: (i, 0)) for _ in range(4)],
    out_shape=[jax.ShapeDtypeStruct((N, HQ), jnp.float32) for _ in range(4)],
)


def _k5_body(a0, a1, a2, a3, y0, y1, y2, y3, dinv_ref, b2_ref, batch_ref,
             wl_ref, bl_ref, out_ref, g_acc):
    i = pl.program_id(0)
    dv = dinv_ref[0, 0, :].reshape(R, 1)
    h = jnp.concatenate(
        [a0[...] + y0[...], a1[...] + y1[...],
         a2[...] + y2[...], a3[...] + y3[...]], axis=1)
    h = jnp.maximum(h * dv + b2_ref[...], 0.0)
    b = batch_ref[0, 0, :]
    onehot = (b[:, None] == lax.broadcasted_iota(jnp.int32, (R, G), 1)
              ).astype(jnp.float32)
    gpart = lax.dot_general(onehot, h, (((0,), (0,)), ((), ())),
                            preferred_element_type=jnp.float32)

    @pl.when(i == 0)
    def _():
        g_acc[...] = gpart

    @pl.when(i > 0)
    def _():
        g_acc[...] += gpart

    @pl.when(i == NB - 1)
    def _():
        out_ref[...] = jnp.dot(g_acc[...], wl_ref[...],
                               preferred_element_type=jnp.float32) + bl_ref[...]


_k5 = pl.pallas_call(
    _k5_body,
    grid=(NB,),
    in_specs=[pl.BlockSpec((R, HQ), lambda i: (i, 0)) for _ in range(8)]
    + [
        pl.BlockSpec((1, 1, R), lambda i: (i, 0, 0)),
        pl.BlockSpec((1, H), lambda i: (0, 0)),
        pl.BlockSpec((1, 1, R), lambda i: (i, 0, 0)),
        pl.BlockSpec((H, C), lambda i: (0, 0)),
        pl.BlockSpec((1, C), lambda i: (0, 0)),
    ],
    out_specs=pl.BlockSpec((G, C), lambda i: (0, 0)),
    out_shape=jax.ShapeDtypeStruct((G, C), jnp.float32),
    scratch_shapes=[pltpu.VMEM((G, H), jnp.float32)],
)


def kernel(x, edge_index, batch, W1, b1, W2, b2, Wl, bl):
    src = edge_index[0]
    dst = edge_index[1]

    ones0 = jnp.ones((K0C,), jnp.float32)

    deg0, deg1 = _deg_kernel(dst, ones0)
    p0 = deg0[:N].reshape(NB, 1, R)
    p1 = deg1[:N].reshape(NB, 1, R)

    dinv3, *y1q = _k1(x, W1, p0, p1)
    a1q = _agg_kernel(*y1q, src, dst)
    y2q = _k3(*[a[:N] for a in a1q], *y1q, dinv3, b1.reshape(1, H), W2)
    a2q = _agg_kernel(*y2q, src, dst)
    out = _k5(*[a[:N] for a in a2q], *y2q, dinv3, b2.reshape(1, H),
              batch.reshape(NB, 1, R), Wl, bl.reshape(1, C))
    return out
